# baseline (speedup 1.0000x reference)
import jax
import jax.numpy as jnp
from jax import lax
from jax.experimental import pallas as pl
from jax.experimental.pallas import tpu as pltpu

_SLOPE = 5.0
_BUDGET = 1.0 / 16.0 - 1.0 / 128.0
_RATIO = 128
_C_LO = 146
_C_HI = 174
_H = 320
_W = 320
_HW = _H * _W
_BPB = 2
_RH = 160

_KS0 = 0
_KS1 = 42
_KS2 = 0x1BD11BDA ^ _KS0 ^ _KS1


def _rotl(x, d):
    return (x << jnp.uint32(d)) | (x >> jnp.uint32(32 - d))


def _threefry2x32_zero_x0(x1_plus_ks1):
    ks0 = jnp.uint32(_KS0)
    ks1 = jnp.uint32(_KS1)
    ks2 = jnp.uint32(_KS2)
    r_a = (13, 15, 26, 6)
    r_b = (17, 29, 16, 24)

    def four_rounds(x0, x1, rots):
        for r in rots:
            x0 = x0 + x1
            x1 = _rotl(x1, r) ^ x0
        return x0, x1

    x0 = x1_plus_ks1
    x1 = _rotl(x1_plus_ks1, 13) ^ x0
    x0, x1 = four_rounds(x0, x1, (15, 26, 6))
    x0 = x0 + ks1
    x1 = x1 + jnp.uint32(_KS2 + 1)
    x0, x1 = four_rounds(x0, x1, r_b)
    x0 = x0 + ks2
    x1 = x1 + jnp.uint32(_KS0 + 2)
    x0, x1 = four_rounds(x0, x1, r_a)
    x0 = x0 + ks0
    x1 = x1 + jnp.uint32(_KS1 + 3)
    x0, x1 = four_rounds(x0, x1, r_b)
    x0 = x0 + ks1
    x1 = x1 + jnp.uint32(_KS2 + 4)
    x0, x1 = four_rounds(x0, x1, r_a)
    x0 = x0 + ks2
    x1 = x1 + jnp.uint32(_KS0 + 5)
    return x0, x1


def _body(w_ref, ks_ref, oks_ref, mask_ref, thr_ref, ju_ref):
    b = pl.program_id(0)
    rj = pl.program_id(1)

    @pl.when((b == 0) & (rj == 0))
    def _prep():
        row = lax.broadcasted_iota(jnp.int32, (_H, _W), 0)
        col = lax.broadcasted_iota(jnp.int32, (_H, _W), 1)
        prob = jax.nn.sigmoid(jnp.float32(_SLOPE) * w_ref[...])
        inside = (row >= _C_LO) & (row < _C_HI) & (col >= _C_LO) & (col < _C_HI)
        prob = jnp.where(inside, jnp.float32(0.0), prob)
        xbar = jnp.mean(prob)
        r = jnp.float32(_BUDGET) / xbar
        beta = (jnp.float32(1.0) - jnp.float32(_BUDGET)) / (jnp.float32(1.0) - xbar)
        mr = jnp.where(
            r <= jnp.float32(1.0),
            prob * r,
            jnp.float32(1.0) - (jnp.float32(1.0) - prob) * beta,
        )
        thr_ref[...] = jnp.ceil(mr * jnp.float32(8388608.0)).astype(jnp.uint32)
        ju_ref[...] = (row * _W + col).astype(jnp.uint32) + jnp.uint32(_KS1)

    r0 = rj * _RH
    ju = ju_ref[pl.ds(r0, _RH), :]
    thr = thr_ref[pl.ds(r0, _RH), :]
    for bi in range(_BPB):
        x1 = ju + lax.convert_element_type((b * _BPB + bi) * _HW, jnp.uint32)
        o0, o1 = _threefry2x32_zero_x0(x1)
        mant = (o0 ^ o1) >> jnp.uint32(9)
        m = (mant < thr).astype(jnp.float32)
        mask_ref[bi] = m
        oks_ref[bi] = ks_ref[bi] * m[None]


@jax.jit
def kernel(kspace, weight):
    B, C = kspace.shape[0], kspace.shape[1]
    oks, mask = pl.pallas_call(
        _body,
        grid=(B // _BPB, _H // _RH),
        in_specs=[
            pl.BlockSpec((_H, _W), lambda b, rj: (0, 0)),
            pl.BlockSpec((_BPB, C, _RH, _W), lambda b, rj: (b, 0, rj, 0)),
        ],
        out_specs=[
            pl.BlockSpec((_BPB, C, _RH, _W), lambda b, rj: (b, 0, rj, 0)),
            pl.BlockSpec((_BPB, _RH, _W), lambda b, rj: (b, rj, 0)),
        ],
        out_shape=[
            jax.ShapeDtypeStruct((B, C, _H, _W), jnp.float32),
            jax.ShapeDtypeStruct((B, _H, _W), jnp.float32),
        ],
        scratch_shapes=[
            pltpu.VMEM((_H, _W), jnp.uint32),
            pltpu.VMEM((_H, _W), jnp.uint32),
        ],
    )(weight, kspace)
    return (
        oks,
        mask,
        jnp.asarray(_RATIO, dtype=jnp.int32),
    )


# speedup vs baseline: 1.0061x; 1.0061x over previous
import jax
import jax.numpy as jnp
from jax import lax
from jax.experimental import pallas as pl
from jax.experimental.pallas import tpu as pltpu

_SLOPE = 5.0
_BUDGET = 1.0 / 16.0 - 1.0 / 128.0
_RATIO = 128
_C_LO = 146
_C_HI = 174
_H = 320
_W = 320
_HW = _H * _W
_BPB = 2

_KS0 = 0
_KS1 = 42
_KS2 = 0x1BD11BDA ^ _KS0 ^ _KS1


def _rotl(x, d):
    return (x << jnp.uint32(d)) | (x >> jnp.uint32(32 - d))


def _threefry2x32_zero_x0(x1_plus_ks1):
    ks0 = jnp.uint32(_KS0)
    ks1 = jnp.uint32(_KS1)
    ks2 = jnp.uint32(_KS2)
    r_a = (13, 15, 26, 6)
    r_b = (17, 29, 16, 24)

    def four_rounds(x0, x1, rots):
        for r in rots:
            x0 = x0 + x1
            x1 = _rotl(x1, r) ^ x0
        return x0, x1

    x0 = x1_plus_ks1
    x1 = _rotl(x1_plus_ks1, 13) ^ x0
    x0, x1 = four_rounds(x0, x1, (15, 26, 6))
    x0 = x0 + ks1
    x1 = x1 + jnp.uint32(_KS2 + 1)
    x0, x1 = four_rounds(x0, x1, r_b)
    x0 = x0 + ks2
    x1 = x1 + jnp.uint32(_KS0 + 2)
    x0, x1 = four_rounds(x0, x1, r_a)
    x0 = x0 + ks0
    x1 = x1 + jnp.uint32(_KS1 + 3)
    x0, x1 = four_rounds(x0, x1, r_b)
    x0 = x0 + ks1
    x1 = x1 + jnp.uint32(_KS2 + 4)
    x0, x1 = four_rounds(x0, x1, r_a)
    x0 = x0 + ks2
    x1 = x1 + jnp.uint32(_KS0 + 5)
    return x0, x1


def _body(w_ref, ks_ref, oks_ref, mask_ref, thr_ref, ju_ref):
    b = pl.program_id(0)

    @pl.when(b == 0)
    def _prep():
        row = lax.broadcasted_iota(jnp.int32, (_H, _W), 0)
        col = lax.broadcasted_iota(jnp.int32, (_H, _W), 1)
        prob = jax.nn.sigmoid(jnp.float32(_SLOPE) * w_ref[...])
        inside = (row >= _C_LO) & (row < _C_HI) & (col >= _C_LO) & (col < _C_HI)
        prob = jnp.where(inside, jnp.float32(0.0), prob)
        xbar = jnp.mean(prob)
        r = jnp.float32(_BUDGET) / xbar
        beta = (jnp.float32(1.0) - jnp.float32(_BUDGET)) / (jnp.float32(1.0) - xbar)
        mr = jnp.where(
            r <= jnp.float32(1.0),
            prob * r,
            jnp.float32(1.0) - (jnp.float32(1.0) - prob) * beta,
        )
        thr_ref[...] = jnp.ceil(mr * jnp.float32(8388608.0)).astype(jnp.uint32)
        ju_ref[...] = (row * _W + col).astype(jnp.uint32) + jnp.uint32(_KS1)

    ju = ju_ref[...]
    thr = thr_ref[...]
    for bi in range(_BPB):
        x1 = ju + lax.convert_element_type((b * _BPB + bi) * _HW, jnp.uint32)
        o0, o1 = _threefry2x32_zero_x0(x1)
        mant = (o0 ^ o1) >> jnp.uint32(9)
        m = (mant < thr).astype(jnp.float32)
        mask_ref[bi] = m
        oks_ref[bi] = ks_ref[bi] * m[None]


@jax.jit
def kernel(kspace, weight):
    B, C = kspace.shape[0], kspace.shape[1]
    oks, mask = pl.pallas_call(
        _body,
        grid=(B // _BPB,),
        in_specs=[
            pl.BlockSpec((_H, _W), lambda b: (0, 0)),
            pl.BlockSpec((_BPB, C, _H, _W), lambda b: (b, 0, 0, 0)),
        ],
        out_specs=[
            pl.BlockSpec((_BPB, C, _H, _W), lambda b: (b, 0, 0, 0)),
            pl.BlockSpec((_BPB, _H, _W), lambda b: (b, 0, 0)),
        ],
        out_shape=[
            jax.ShapeDtypeStruct((B, C, _H, _W), jnp.float32),
            jax.ShapeDtypeStruct((B, _H, _W), jnp.float32),
        ],
        scratch_shapes=[
            pltpu.VMEM((_H, _W), jnp.uint32),
            pltpu.VMEM((_H, _W), jnp.uint32),
        ],
    )(weight, kspace)
    return (
        oks,
        mask,
        jnp.asarray(_RATIO, dtype=jnp.int32),
    )


# speedup vs baseline: 1.0194x; 1.0132x over previous
import jax
import jax.numpy as jnp
from jax import lax
from jax.experimental import pallas as pl
from jax.experimental.pallas import tpu as pltpu

_SLOPE = 5.0
_BUDGET = 1.0 / 16.0 - 1.0 / 128.0
_RATIO = 128
_C_LO = 146
_C_HI = 174
_H = 320
_W = 320
_HW = _H * _W
_BPB = 2

_KS0 = 0
_KS1 = 42
_KS2 = 0x1BD11BDA ^ _KS0 ^ _KS1


def _rotl(x, d):
    return (x << jnp.uint32(d)) | (x >> jnp.uint32(32 - d))


def _threefry2x32_zero_x0(x1_plus_ks1):
    ks0 = jnp.uint32(_KS0)
    ks1 = jnp.uint32(_KS1)
    ks2 = jnp.uint32(_KS2)
    r_a = (13, 15, 26, 6)
    r_b = (17, 29, 16, 24)

    def four_rounds(x0, x1, rots):
        for r in rots:
            x0 = x0 + x1
            x1 = _rotl(x1, r) ^ x0
        return x0, x1

    x0 = x1_plus_ks1
    x1 = _rotl(x1_plus_ks1, 13) ^ x0
    x0, x1 = four_rounds(x0, x1, (15, 26, 6))
    x0 = x0 + ks1
    x1 = x1 + jnp.uint32(_KS2 + 1)
    x0, x1 = four_rounds(x0, x1, r_b)
    x0 = x0 + ks2
    x1 = x1 + jnp.uint32(_KS0 + 2)
    x0, x1 = four_rounds(x0, x1, r_a)
    x0 = x0 + ks0
    x1 = x1 + jnp.uint32(_KS1 + 3)
    x0, x1 = four_rounds(x0, x1, r_b)
    x0 = x0 + ks1
    x1 = x1 + jnp.uint32(_KS2 + 4)
    x0, x1 = four_rounds(x0, x1, r_a)
    x0 = x0 + ks2
    x1 = x1 + jnp.uint32(_KS0 + 5)
    return x0, x1


def _body(w_ref, ks_ref, oks_ref, mask_ref, thr_ref, ju_ref):
    b = pl.program_id(0)

    @pl.when(b == 0)
    def _prep():
        row = lax.broadcasted_iota(jnp.int32, (_H, _W), 0)
        col = lax.broadcasted_iota(jnp.int32, (_H, _W), 1)
        prob = jax.nn.sigmoid(jnp.float32(_SLOPE) * w_ref[...])
        inside = (row >= _C_LO) & (row < _C_HI) & (col >= _C_LO) & (col < _C_HI)
        prob = jnp.where(inside, jnp.float32(0.0), prob)
        xbar = jnp.mean(prob)
        r = jnp.float32(_BUDGET) / xbar
        beta = (jnp.float32(1.0) - jnp.float32(_BUDGET)) / (jnp.float32(1.0) - xbar)
        mr = jnp.where(
            r <= jnp.float32(1.0),
            prob * r,
            jnp.float32(1.0) - (jnp.float32(1.0) - prob) * beta,
        )
        thr_ref[...] = jnp.ceil(mr * jnp.float32(8388608.0)).astype(jnp.uint32)
        ju_ref[...] = (row * _W + col).astype(jnp.uint32) + jnp.uint32(_KS1)

    for bi in range(_BPB):
        base = lax.convert_element_type((b * _BPB + bi) * _HW, jnp.uint32)
        for s in range(2):
            sl = pl.ds(s * (_H // 2), _H // 2)
            x1 = ju_ref[sl, :] + base
            o0, o1 = _threefry2x32_zero_x0(x1)
            mant = (o0 ^ o1) >> jnp.uint32(9)
            m = (mant < thr_ref[sl, :]).astype(jnp.float32)
            mask_ref[bi, sl] = m
            oks_ref[bi, 0, sl] = ks_ref[bi, 0, sl] * m
            oks_ref[bi, 1, sl] = ks_ref[bi, 1, sl] * m


@jax.jit
def kernel(kspace, weight):
    B, C = kspace.shape[0], kspace.shape[1]
    oks, mask = pl.pallas_call(
        _body,
        grid=(B // _BPB,),
        in_specs=[
            pl.BlockSpec((_H, _W), lambda b: (0, 0)),
            pl.BlockSpec((_BPB, C, _H, _W), lambda b: (b, 0, 0, 0)),
        ],
        out_specs=[
            pl.BlockSpec((_BPB, C, _H, _W), lambda b: (b, 0, 0, 0)),
            pl.BlockSpec((_BPB, _H, _W), lambda b: (b, 0, 0)),
        ],
        out_shape=[
            jax.ShapeDtypeStruct((B, C, _H, _W), jnp.float32),
            jax.ShapeDtypeStruct((B, _H, _W), jnp.float32),
        ],
        scratch_shapes=[
            pltpu.VMEM((_H, _W), jnp.uint32),
            pltpu.VMEM((_H, _W), jnp.uint32),
        ],
    )(weight, kspace)
    return (
        oks,
        mask,
        jnp.asarray(_RATIO, dtype=jnp.int32),
    )


# speedup vs baseline: 1.0201x; 1.0007x over previous
import jax
import jax.numpy as jnp
from jax import lax
from jax.experimental import pallas as pl
from jax.experimental.pallas import tpu as pltpu

_SLOPE = 5.0
_BUDGET = 1.0 / 16.0 - 1.0 / 128.0
_RATIO = 128
_C_LO = 146
_C_HI = 174
_H = 320
_W = 320
_HW = _H * _W
_BPB = 2
_NCH = 4

_KS0 = 0
_KS1 = 42
_KS2 = 0x1BD11BDA ^ _KS0 ^ _KS1


def _rotl(x, d):
    return (x << jnp.uint32(d)) | (x >> jnp.uint32(32 - d))


def _threefry2x32_zero_x0(x1_plus_ks1):
    ks0 = jnp.uint32(_KS0)
    ks1 = jnp.uint32(_KS1)
    ks2 = jnp.uint32(_KS2)
    r_a = (13, 15, 26, 6)
    r_b = (17, 29, 16, 24)

    def four_rounds(x0, x1, rots):
        for r in rots:
            x0 = x0 + x1
            x1 = _rotl(x1, r) ^ x0
        return x0, x1

    x0 = x1_plus_ks1
    x1 = _rotl(x1_plus_ks1, 13) ^ x0
    x0, x1 = four_rounds(x0, x1, (15, 26, 6))
    x0 = x0 + ks1
    x1 = x1 + jnp.uint32(_KS2 + 1)
    x0, x1 = four_rounds(x0, x1, r_b)
    x0 = x0 + ks2
    x1 = x1 + jnp.uint32(_KS0 + 2)
    x0, x1 = four_rounds(x0, x1, r_a)
    x0 = x0 + ks0
    x1 = x1 + jnp.uint32(_KS1 + 3)
    x0, x1 = four_rounds(x0, x1, r_b)
    x0 = x0 + ks1
    x1 = x1 + jnp.uint32(_KS2 + 4)
    x0, x1 = four_rounds(x0, x1, r_a)
    x0 = x0 + ks2
    x1 = x1 + jnp.uint32(_KS0 + 5)
    return x0, x1


def _body(w_ref, ks_ref, oks_ref, mask_ref, thr_ref, ju_ref):
    b = pl.program_id(0)

    @pl.when(b == 0)
    def _prep():
        row = lax.broadcasted_iota(jnp.int32, (_H, _W), 0)
        col = lax.broadcasted_iota(jnp.int32, (_H, _W), 1)
        prob = jax.nn.sigmoid(jnp.float32(_SLOPE) * w_ref[...])
        inside = (row >= _C_LO) & (row < _C_HI) & (col >= _C_LO) & (col < _C_HI)
        prob = jnp.where(inside, jnp.float32(0.0), prob)
        xbar = jnp.mean(prob)
        r = jnp.float32(_BUDGET) / xbar
        beta = (jnp.float32(1.0) - jnp.float32(_BUDGET)) / (jnp.float32(1.0) - xbar)
        mr = jnp.where(
            r <= jnp.float32(1.0),
            prob * r,
            jnp.float32(1.0) - (jnp.float32(1.0) - prob) * beta,
        )
        thr_ref[...] = jnp.ceil(mr * jnp.float32(8388608.0)).astype(jnp.uint32)
        ju_ref[...] = (row * _W + col).astype(jnp.uint32) + jnp.uint32(_KS1)

    for bi in range(_BPB):
        base = lax.convert_element_type((b * _BPB + bi) * _HW, jnp.uint32)
        for s in range(_NCH):
            sl = pl.ds(s * (_H // _NCH), _H // _NCH)
            x1 = ju_ref[sl, :] + base
            o0, o1 = _threefry2x32_zero_x0(x1)
            mant = (o0 ^ o1) >> jnp.uint32(9)
            m = (mant < thr_ref[sl, :]).astype(jnp.float32)
            mask_ref[bi, sl] = m
            oks_ref[bi, 0, sl] = ks_ref[bi, 0, sl] * m
            oks_ref[bi, 1, sl] = ks_ref[bi, 1, sl] * m


@jax.jit
def kernel(kspace, weight):
    B, C = kspace.shape[0], kspace.shape[1]
    oks, mask = pl.pallas_call(
        _body,
        grid=(B // _BPB,),
        in_specs=[
            pl.BlockSpec((_H, _W), lambda b: (0, 0)),
            pl.BlockSpec((_BPB, C, _H, _W), lambda b: (b, 0, 0, 0)),
        ],
        out_specs=[
            pl.BlockSpec((_BPB, C, _H, _W), lambda b: (b, 0, 0, 0)),
            pl.BlockSpec((_BPB, _H, _W), lambda b: (b, 0, 0)),
        ],
        out_shape=[
            jax.ShapeDtypeStruct((B, C, _H, _W), jnp.float32),
            jax.ShapeDtypeStruct((B, _H, _W), jnp.float32),
        ],
        scratch_shapes=[
            pltpu.VMEM((_H, _W), jnp.uint32),
            pltpu.VMEM((_H, _W), jnp.uint32),
        ],
    )(weight, kspace)
    return (
        oks,
        mask,
        jnp.asarray(_RATIO, dtype=jnp.int32),
    )


# speedup vs baseline: 1.0202x; 1.0001x over previous
import jax
import jax.numpy as jnp
from jax import lax
from jax.experimental import pallas as pl
from jax.experimental.pallas import tpu as pltpu

_SLOPE = 5.0
_BUDGET = 1.0 / 16.0 - 1.0 / 128.0
_RATIO = 128
_C_LO = 146
_C_HI = 174
_H = 320
_W = 320
_HW = _H * _W
_BPB = 2
_NCH = 8

_KS0 = 0
_KS1 = 42
_KS2 = 0x1BD11BDA ^ _KS0 ^ _KS1


def _rotl(x, d):
    return (x << jnp.uint32(d)) | (x >> jnp.uint32(32 - d))


def _threefry2x32_zero_x0(x1_plus_ks1):
    ks0 = jnp.uint32(_KS0)
    ks1 = jnp.uint32(_KS1)
    ks2 = jnp.uint32(_KS2)
    r_a = (13, 15, 26, 6)
    r_b = (17, 29, 16, 24)

    def four_rounds(x0, x1, rots):
        for r in rots:
            x0 = x0 + x1
            x1 = _rotl(x1, r) ^ x0
        return x0, x1

    x0 = x1_plus_ks1
    x1 = _rotl(x1_plus_ks1, 13) ^ x0
    x0, x1 = four_rounds(x0, x1, (15, 26, 6))
    x0 = x0 + ks1
    x1 = x1 + jnp.uint32(_KS2 + 1)
    x0, x1 = four_rounds(x0, x1, r_b)
    x0 = x0 + ks2
    x1 = x1 + jnp.uint32(_KS0 + 2)
    x0, x1 = four_rounds(x0, x1, r_a)
    x0 = x0 + ks0
    x1 = x1 + jnp.uint32(_KS1 + 3)
    x0, x1 = four_rounds(x0, x1, r_b)
    x0 = x0 + ks1
    x1 = x1 + jnp.uint32(_KS2 + 4)
    x0, x1 = four_rounds(x0, x1, r_a)
    x0 = x0 + ks2
    x1 = x1 + jnp.uint32(_KS0 + 5)
    return x0, x1


def _body(w_ref, ks_ref, oks_ref, mask_ref, thr_ref, ju_ref):
    b = pl.program_id(0)

    @pl.when(b == 0)
    def _prep():
        row = lax.broadcasted_iota(jnp.int32, (_H, _W), 0)
        col = lax.broadcasted_iota(jnp.int32, (_H, _W), 1)
        prob = jax.nn.sigmoid(jnp.float32(_SLOPE) * w_ref[...])
        inside = (row >= _C_LO) & (row < _C_HI) & (col >= _C_LO) & (col < _C_HI)
        prob = jnp.where(inside, jnp.float32(0.0), prob)
        xbar = jnp.mean(prob)
        r = jnp.float32(_BUDGET) / xbar
        beta = (jnp.float32(1.0) - jnp.float32(_BUDGET)) / (jnp.float32(1.0) - xbar)
        mr = jnp.where(
            r <= jnp.float32(1.0),
            prob * r,
            jnp.float32(1.0) - (jnp.float32(1.0) - prob) * beta,
        )
        thr_ref[...] = jnp.ceil(mr * jnp.float32(8388608.0)).astype(jnp.uint32)
        ju_ref[...] = (row * _W + col).astype(jnp.uint32) + jnp.uint32(_KS1)

    for bi in range(_BPB):
        base = lax.convert_element_type((b * _BPB + bi) * _HW, jnp.uint32)
        for s in range(_NCH):
            sl = pl.ds(s * (_H // _NCH), _H // _NCH)
            x1 = ju_ref[sl, :] + base
            o0, o1 = _threefry2x32_zero_x0(x1)
            mant = (o0 ^ o1) >> jnp.uint32(9)
            m = (mant < thr_ref[sl, :]).astype(jnp.float32)
            mask_ref[bi, sl] = m
            oks_ref[bi, 0, sl] = ks_ref[bi, 0, sl] * m
            oks_ref[bi, 1, sl] = ks_ref[bi, 1, sl] * m


@jax.jit
def kernel(kspace, weight):
    B, C = kspace.shape[0], kspace.shape[1]
    oks, mask = pl.pallas_call(
        _body,
        grid=(B // _BPB,),
        in_specs=[
            pl.BlockSpec((_H, _W), lambda b: (0, 0)),
            pl.BlockSpec((_BPB, C, _H, _W), lambda b: (b, 0, 0, 0)),
        ],
        out_specs=[
            pl.BlockSpec((_BPB, C, _H, _W), lambda b: (b, 0, 0, 0)),
            pl.BlockSpec((_BPB, _H, _W), lambda b: (b, 0, 0)),
        ],
        out_shape=[
            jax.ShapeDtypeStruct((B, C, _H, _W), jnp.float32),
            jax.ShapeDtypeStruct((B, _H, _W), jnp.float32),
        ],
        scratch_shapes=[
            pltpu.VMEM((_H, _W), jnp.uint32),
            pltpu.VMEM((_H, _W), jnp.uint32),
        ],
    )(weight, kspace)
    return (
        oks,
        mask,
        jnp.asarray(_RATIO, dtype=jnp.int32),
    )
